# Initial kernel scaffold; baseline (speedup 1.0000x reference)
import jax
import jax.numpy as jnp
from jax.experimental import pallas as pl


def kernel(x, patterns, quantize_edges):
    raise NotImplementedError("write your pallas kernel here")


# speedup vs baseline: 10.4624x; 10.4624x over previous
import functools

import jax
import jax.numpy as jnp
from jax import lax
from jax.experimental import pallas as pl
from jax.experimental.pallas import tpu as pltpu

_N_BINS = 8
_P = 1024
_D = 64


def _match_kernel(edges_ref, x_ref, pat_ref, best_ref, score_ref):
    xb = x_ref[0]
    pb = pat_ref[...]

    qx = jnp.zeros(xb.shape, jnp.float32)
    qp = jnp.zeros(pb.shape, jnp.float32)
    for i in range(7):
        e = edges_ref[i]
        qx = qx + (xb > e).astype(jnp.float32)
        qp = qp + (pb > e).astype(jnp.float32)

    acc = jnp.zeros((_P, xb.shape[0]), jnp.float32)
    for b in range(_N_BINS):
        a = (qx == b).astype(jnp.bfloat16)
        p1 = (qp == b).astype(jnp.bfloat16)
        acc = acc + lax.dot_general(
            p1, a, (((1,), (1,)), ((), ())),
            preferred_element_type=jnp.float32)

    counts = acc.astype(jnp.int32)
    rev = (_P - 1) - lax.broadcasted_iota(jnp.int32, acc.shape, 0)
    val = counts * _P + rev
    m = jnp.max(val, axis=0)
    best = (_P - 1) - (m & (_P - 1))
    score = (m >> 10).astype(jnp.float32) * (1.0 / _D)
    best_ref[0, 0, :] = best
    score_ref[0, 0, :] = score


def kernel(x, patterns, quantize_edges):
    B, S, D = x.shape
    grid = (B,)
    best, score = pl.pallas_call(
        _match_kernel,
        grid=grid,
        in_specs=[
            pl.BlockSpec(memory_space=pltpu.SMEM),
            pl.BlockSpec((1, S, D), lambda i: (i, 0, 0)),
            pl.BlockSpec((_P, D), lambda i: (0, 0)),
        ],
        out_specs=[
            pl.BlockSpec((1, 1, S), lambda i: (i, 0, 0)),
            pl.BlockSpec((1, 1, S), lambda i: (i, 0, 0)),
        ],
        out_shape=[
            jax.ShapeDtypeStruct((B, 1, S), jnp.int32),
            jax.ShapeDtypeStruct((B, 1, S), jnp.float32),
        ],
    )(quantize_edges, x, patterns)
    return best.reshape(B, S), score.reshape(B, S)
